# baseline (speedup 1.0000x reference)
import functools

import jax
import jax.numpy as jnp
from jax import lax
from jax.experimental import pallas as pl
from jax.experimental.pallas import tpu as pltpu
from jax.experimental.pallas import tpu_sc as plsc

NUM_LAYERS = 5
EMB = 128
HC = 64
N = 10000
E = 320000

NC = 2
NS = 16
K = 128
NPAD = 10240
ROWS_PER_TILE = NPAD // NS
EPAD = 327680
ET = EPAD // NS
NCHUNK = ET // K
NBUF = 4

_MESH = plsc.VectorSubcoreMesh(
    core_axis_name="c", subcore_axis_name="s", num_cores=NC, num_subcores=NS)


def _ring(tab_sh, agg_sh, src_t, dst_t, rows_v, gsem, ssem, nch):
    def fire(i, b):
        pltpu.async_copy(tab_sh.at[src_t.at[i]], rows_v.at[b], gsem[b])

    for b in range(NBUF):
        fire(b, b)

    def outer(j, carry):
        base_i = j * NBUF
        for b in range(NBUF):
            i = base_i + b
            pltpu.make_async_copy(
                tab_sh.at[src_t.at[i]], rows_v.at[b], gsem[b]).wait()
            pltpu.async_copy(
                rows_v.at[b], agg_sh.at[dst_t.at[i]], ssem[b], add=True)
        for b in range(NBUF):
            i = base_i + b
            pltpu.make_async_copy(
                rows_v.at[b], agg_sh.at[dst_t.at[i]], ssem[b]).wait()
            nxt = base_i + NBUF + b

            @pl.when(nxt < nch)
            def _():
                fire(nxt, b)
        return carry

    lax.fori_loop(0, nch // NBUF, outer, 0)


TREP = 2048
ECH = NCHUNK // 2


def _cnt_sp_body(tab_hbm, ind_hbm, dst_hbm, z_hbm, out_hbm,
                 src_t, dst_t, rows_v, *rest):
    gsem = rest[0:NBUF]
    ssem = rest[NBUF:2 * NBUF]
    agg_sh = rest[2 * NBUF]
    tab_sh = rest[2 * NBUF + 1]
    c = lax.axis_index("c")
    s = lax.axis_index("s")
    r0 = s * ROWS_PER_TILE
    pltpu.sync_copy(z_hbm, agg_sh.at[pl.ds(r0, ROWS_PER_TILE)])
    t0 = s * (TREP // NS)
    pltpu.sync_copy(tab_hbm.at[pl.ds(t0, TREP // NS)],
                    tab_sh.at[pl.ds(t0, TREP // NS)])
    cbase = (c * NS + s) * ECH
    pltpu.sync_copy(ind_hbm.at[pl.ds(cbase, ECH)], src_t)
    pltpu.sync_copy(dst_hbm.at[pl.ds(cbase, ECH)], dst_t)
    plsc.subcore_barrier()

    _ring(tab_sh, agg_sh, src_t, dst_t, rows_v, gsem, ssem, ECH)
    plsc.subcore_barrier()
    pltpu.sync_copy(agg_sh.at[pl.ds(r0, ROWS_PER_TILE)],
                    out_hbm.at[c, pl.ds(r0, ROWS_PER_TILE)])


_seg_es = functools.partial(
    pl.kernel, _cnt_sp_body,
    out_type=jax.ShapeDtypeStruct((NC, NPAD, HC), jnp.float32),
    mesh=_MESH,
    scratch_types=([pltpu.VMEM((ECH, K), jnp.int32),
                    pltpu.VMEM((ECH, K), jnp.int32),
                    pltpu.VMEM((NBUF, K, HC), jnp.float32)]
                   + [pltpu.SemaphoreType.DMA] * (2 * NBUF)
                   + [pltpu.VMEM_SHARED((NPAD, HC), jnp.float32),
                      pltpu.VMEM_SHARED((TREP, HC), jnp.float32)]),
    compiler_params=pltpu.CompilerParams(use_tc_tiling_on_sc=False),
)()

TROWS = N // NS
NPH = 4
PCH = NCHUNK // NPH


def _seg_sp_body(tab_hbm, ind_hbm, dst_hbm, z_hbm, out_hbm,
                 src_t, dst_t, rows_v, *rest):
    gsem = rest[0:NBUF]
    ssem = rest[NBUF:2 * NBUF]
    agg_sh = rest[2 * NBUF]
    tab_sh = rest[2 * NBUF + 1]
    c = lax.axis_index("c")
    s = lax.axis_index("s")
    r0 = s * ROWS_PER_TILE
    pltpu.sync_copy(z_hbm, agg_sh.at[pl.ds(r0, ROWS_PER_TILE)])
    t0 = s * TROWS
    pltpu.sync_copy(tab_hbm.at[c, pl.ds(t0, TROWS)], tab_sh.at[pl.ds(t0, TROWS)])
    plsc.subcore_barrier()

    for ph in range(NPH):
        cbase = s * NCHUNK + ph * PCH
        pltpu.sync_copy(ind_hbm.at[pl.ds(cbase, PCH)], src_t)
        pltpu.sync_copy(dst_hbm.at[pl.ds(cbase, PCH)], dst_t)
        _ring(tab_sh, agg_sh, src_t, dst_t, rows_v, gsem, ssem, PCH)

    plsc.subcore_barrier()
    pltpu.sync_copy(agg_sh.at[pl.ds(r0, ROWS_PER_TILE)],
                    out_hbm.at[c, pl.ds(r0, ROWS_PER_TILE)])


_seg = functools.partial(
    pl.kernel, _seg_sp_body,
    out_type=jax.ShapeDtypeStruct((NC, NPAD, HC), jnp.float32),
    mesh=_MESH,
    scratch_types=([pltpu.VMEM((PCH, K), jnp.int32),
                    pltpu.VMEM((PCH, K), jnp.int32),
                    pltpu.VMEM((NBUF, K, HC), jnp.float32)]
                   + [pltpu.SemaphoreType.DMA] * (2 * NBUF)
                   + [pltpu.VMEM_SHARED((NPAD, HC), jnp.float32),
                      pltpu.VMEM_SHARED((N, HC), jnp.float32)]),
    compiler_params=pltpu.CompilerParams(use_tc_tiling_on_sc=False),
)()


def _init_body(x0h_ref, x1h_ref, a1_ref, a2_ref, o_ref):
    h = (jnp.dot(x0h_ref[...], a1_ref[...], preferred_element_type=jnp.float32)
         + jnp.dot(x1h_ref[...], a2_ref[...], preferred_element_type=jnp.float32))
    o_ref[0] = h[:, :HC]
    o_ref[1] = h[:, HC:]


_init_tc = pl.pallas_call(
    _init_body, out_shape=jax.ShapeDtypeStruct((NC, N, HC), jnp.float32))


def _layer_body(head, p_ref, cnt_ref, h_ref, ce_ref, cl_ref,
                w1_ref, b1_ref, w2_ref, b2_ref, g_ref, bt_ref,
                wm1_ref, bm1_ref, wm2_ref, bm2_ref, wm3_ref, bm3_ref, o_ref):
    cnt = cnt_ref[0, :N, :] + cnt_ref[1, :N, :]
    p = jnp.concatenate([p_ref[0, :N, :], p_ref[1, :N, :]], axis=1)
    h = jnp.concatenate([h_ref[0], h_ref[1]], axis=1)
    agg = (p + h
           + jnp.dot(cnt, ce_ref[...], preferred_element_type=jnp.float32)
           + cl_ref[...])
    hid = jnp.maximum(
        jnp.dot(agg, w1_ref[...], preferred_element_type=jnp.float32)
        + b1_ref[...], 0.0)
    hn = (jnp.dot(hid, w2_ref[...], preferred_element_type=jnp.float32)
          + b2_ref[...])
    mean = jnp.mean(hn, axis=0, keepdims=True)
    var = jnp.mean((hn - mean) ** 2, axis=0, keepdims=True)
    hn = (hn - mean) * lax.rsqrt(var + 1e-5) * g_ref[...] + bt_ref[...]
    if head:
        z = jnp.maximum(
            jnp.dot(hn, wm1_ref[...], preferred_element_type=jnp.float32)
            + bm1_ref[...], 0.0)
        z = jnp.maximum(
            jnp.dot(z, wm2_ref[...], preferred_element_type=jnp.float32)
            + bm2_ref[...], 0.0)
        logit = (jnp.dot(z, wm3_ref[...], preferred_element_type=jnp.float32)
                 + bm3_ref[...])
        o_ref[...] = jax.nn.sigmoid(logit)
    else:
        hn = jnp.maximum(hn, 0.0)
        o_ref[0] = hn[:, :HC]
        o_ref[1] = hn[:, HC:]


_layer_tc = pl.pallas_call(
    functools.partial(_layer_body, False),
    out_shape=jax.ShapeDtypeStruct((NC, N, HC), jnp.float32))

_layer_tc_head = pl.pallas_call(
    functools.partial(_layer_body, True),
    out_shape=jax.ShapeDtypeStruct((N, 1), jnp.float32))


def kernel(x, edge_index, edge_attr, atom_emb1, atom_emb2, W1, b1, W2, b2,
           bond_emb, bond_dir_emb, bn_gamma, bn_beta,
           Wm1, bm1, Wm2, bm2, Wm3, bm3):
    f32 = jnp.float32
    src = edge_index[0].astype(jnp.int32)
    dst = edge_index[1].astype(jnp.int32)
    combo = (edge_attr[:, 0] * 3 + edge_attr[:, 1]).astype(jnp.int32)
    pad = EPAD - E
    src_p = jnp.concatenate([src, jnp.zeros((pad,), jnp.int32)]
                            ).reshape(EPAD // K, K)
    dst_p = jnp.concatenate([dst, jnp.full((pad,), NPAD - 1, jnp.int32)]
                            ).reshape(EPAD // K, K)
    combo_p = jnp.concatenate([combo, jnp.zeros((pad,), jnp.int32)]
                              ).reshape(EPAD // K, K)
    combo_p = combo_p + 16 * jnp.arange(K, dtype=jnp.int32)[None, :]
    id_rep = jnp.tile(jnp.eye(16, HC, dtype=f32), (K, 1))
    z_agg = jnp.zeros((ROWS_PER_TILE, HC), f32)

    oh_iota = jnp.arange(8, dtype=x.dtype)[None, :]
    x0h = (x[:, 0:1] == oh_iota).astype(f32)
    x1h = (x[:, 1:2] == oh_iota).astype(f32)
    a1p = jnp.zeros((8, EMB), f32).at[:3].set(atom_emb1[:3])
    a2p = jnp.zeros((8, EMB), f32).at[:3].set(atom_emb2[:3])
    h = _init_tc(x0h, x1h, a1p, a2p)

    cnts = _seg_es(id_rep, combo_p, dst_p, z_agg)

    ia = jnp.repeat(jnp.arange(3), 3)
    ib = jnp.tile(jnp.arange(3), 3)
    b1_2d = b1.reshape(NUM_LAYERS, 1, 2 * EMB)
    b2_2d = b2.reshape(NUM_LAYERS, 1, EMB)
    g_2d = bn_gamma.reshape(NUM_LAYERS, 1, EMB)
    bt_2d = bn_beta.reshape(NUM_LAYERS, 1, EMB)
    bm1_2d = bm1.reshape(1, 2 * EMB)
    bm2_2d = bm2.reshape(1, EMB)
    bm3_2d = bm3.reshape(1, 1)

    out = None
    for l in range(NUM_LAYERS):
        parts = _seg(h, src_p, dst_p, z_agg)
        ce_l = jnp.zeros((HC, EMB), f32).at[:9].set(
            bond_emb[l][ia] + bond_dir_emb[l][ib])
        cl = (bond_emb[l][4] + bond_dir_emb[l][0]).reshape(1, EMB)
        fn = _layer_tc if l < NUM_LAYERS - 1 else _layer_tc_head
        out = fn(parts, cnts, h, ce_l, cl,
                 W1[l], b1_2d[l], W2[l], b2_2d[l], g_2d[l], bt_2d[l],
                 Wm1, bm1_2d, Wm2, bm2_2d, Wm3, bm3_2d)
        if l < NUM_LAYERS - 1:
            h = out
    return out.reshape(-1)


# speedup vs baseline: 1.1141x; 1.1141x over previous
import functools

import jax
import jax.numpy as jnp
from jax import lax
from jax.experimental import pallas as pl
from jax.experimental.pallas import tpu as pltpu
from jax.experimental.pallas import tpu_sc as plsc

NUM_LAYERS = 5
EMB = 128
HC = 64
N = 10000
E = 320000

NC = 2
NS = 16
K = 128
NPAD = 10240
ROWS_PER_TILE = NPAD // NS
EPAD = 327680
ET = EPAD // NS
NCHUNK = ET // K
NBUF = 4

_MESH = plsc.VectorSubcoreMesh(
    core_axis_name="c", subcore_axis_name="s", num_cores=NC, num_subcores=NS)


def _seg_body(edge_split, tab_hbm, ind_hbm, dst_hbm, z_hbm, out_hbm,
              src_t, dst_t, rows_v, *rest):
    nch = NCHUNK // 2 if edge_split else NCHUNK
    gsem = rest[0:NBUF]
    ssem = rest[NBUF:2 * NBUF]
    agg_sh = rest[2 * NBUF]
    c = lax.axis_index("c")
    s = lax.axis_index("s")
    r0 = s * ROWS_PER_TILE
    pltpu.sync_copy(z_hbm, agg_sh.at[pl.ds(r0, ROWS_PER_TILE)])

    cbase = ((c * NS + s) * nch) if edge_split else (s * nch)
    pltpu.sync_copy(ind_hbm.at[pl.ds(cbase, nch)], src_t)
    pltpu.sync_copy(dst_hbm.at[pl.ds(cbase, nch)], dst_t)
    plsc.subcore_barrier()

    tab_c = tab_hbm if edge_split else tab_hbm.at[c]

    def fire(i, b):
        pltpu.async_copy(tab_c.at[src_t.at[i]], rows_v.at[b], gsem[b])

    for b in range(NBUF):
        fire(b, b)

    def outer(j, carry):
        base_i = j * NBUF
        for b in range(NBUF):
            i = base_i + b
            pltpu.make_async_copy(
                tab_c.at[src_t.at[i]], rows_v.at[b], gsem[b]).wait()
            pltpu.async_copy(
                rows_v.at[b], agg_sh.at[dst_t.at[i]], ssem[b], add=True)
        for b in range(NBUF):
            i = base_i + b
            pltpu.make_async_copy(
                rows_v.at[b], agg_sh.at[dst_t.at[i]], ssem[b]).wait()
            nxt = base_i + NBUF + b

            @pl.when(nxt < nch)
            def _():
                fire(nxt, b)
        return carry

    lax.fori_loop(0, nch // NBUF, outer, 0)
    plsc.subcore_barrier()
    pltpu.sync_copy(agg_sh.at[pl.ds(r0, ROWS_PER_TILE)],
                    out_hbm.at[c, pl.ds(r0, ROWS_PER_TILE)])


def _make_seg(edge_split):
    nch = NCHUNK // 2 if edge_split else NCHUNK
    return functools.partial(
        pl.kernel, functools.partial(_seg_body, edge_split),
        out_type=jax.ShapeDtypeStruct((NC, NPAD, HC), jnp.float32),
        mesh=_MESH,
        scratch_types=([pltpu.VMEM((nch, K), jnp.int32),
                        pltpu.VMEM((nch, K), jnp.int32),
                        pltpu.VMEM((NBUF, K, HC), jnp.float32)]
                       + [pltpu.SemaphoreType.DMA] * (2 * NBUF)
                       + [pltpu.VMEM_SHARED((NPAD, HC), jnp.float32)]),
        compiler_params=pltpu.CompilerParams(use_tc_tiling_on_sc=False),
    )()


_seg_es = _make_seg(True)

TROWS = N // NS
NPH = 4
PCH = NCHUNK // NPH


def _seg_sp_body(tab_hbm, ind_hbm, dst_hbm, z_hbm, out_hbm,
                 src_t, dst_t, rows_v, *rest):
    gsem = rest[0:NBUF]
    ssem = rest[NBUF:2 * NBUF]
    agg_sh = rest[2 * NBUF]
    tab_sh = rest[2 * NBUF + 1]
    c = lax.axis_index("c")
    s = lax.axis_index("s")
    r0 = s * ROWS_PER_TILE
    pltpu.sync_copy(z_hbm, agg_sh.at[pl.ds(r0, ROWS_PER_TILE)])
    t0 = s * TROWS
    pltpu.sync_copy(tab_hbm.at[pl.ds(t0, TROWS), pl.ds(c * HC, HC)],
                    tab_sh.at[pl.ds(t0, TROWS)])
    plsc.subcore_barrier()

    def fire(i, b):
        pltpu.async_copy(tab_sh.at[src_t.at[i]], rows_v.at[b], gsem[b])

    def outer(j, carry):
        base_i = j * NBUF
        for b in range(NBUF):
            i = base_i + b
            pltpu.make_async_copy(
                tab_sh.at[src_t.at[i]], rows_v.at[b], gsem[b]).wait()
            pltpu.async_copy(
                rows_v.at[b], agg_sh.at[dst_t.at[i]], ssem[b], add=True)
        for b in range(NBUF):
            i = base_i + b
            pltpu.make_async_copy(
                rows_v.at[b], agg_sh.at[dst_t.at[i]], ssem[b]).wait()
            nxt = base_i + NBUF + b

            @pl.when(nxt < PCH)
            def _():
                fire(nxt, b)
        return carry

    for ph in range(NPH):
        cbase = s * NCHUNK + ph * PCH
        pltpu.sync_copy(ind_hbm.at[pl.ds(cbase, PCH)], src_t)
        pltpu.sync_copy(dst_hbm.at[pl.ds(cbase, PCH)], dst_t)
        for b in range(NBUF):
            fire(b, b)
        lax.fori_loop(0, PCH // NBUF, outer, 0)

    plsc.subcore_barrier()
    pltpu.sync_copy(agg_sh.at[pl.ds(r0, ROWS_PER_TILE)],
                    out_hbm.at[pl.ds(r0, ROWS_PER_TILE), pl.ds(c * HC, HC)])


_seg = functools.partial(
    pl.kernel, _seg_sp_body,
    out_type=jax.ShapeDtypeStruct((NPAD, EMB), jnp.float32),
    mesh=_MESH,
    scratch_types=([pltpu.VMEM((PCH, K), jnp.int32),
                    pltpu.VMEM((PCH, K), jnp.int32),
                    pltpu.VMEM((NBUF, K, HC), jnp.float32)]
                   + [pltpu.SemaphoreType.DMA] * (2 * NBUF)
                   + [pltpu.VMEM_SHARED((NPAD, HC), jnp.float32),
                      pltpu.VMEM_SHARED((N, HC), jnp.float32)]),
    compiler_params=pltpu.CompilerParams(use_tc_tiling_on_sc=False),
)()


def _init_body(x0h_ref, x1h_ref, a1_ref, a2_ref, o_ref):
    o_ref[...] = (
        jnp.dot(x0h_ref[...], a1_ref[...], preferred_element_type=jnp.float32)
        + jnp.dot(x1h_ref[...], a2_ref[...], preferred_element_type=jnp.float32))


_init_tc = pl.pallas_call(
    _init_body, out_shape=jax.ShapeDtypeStruct((N, EMB), jnp.float32))


def _layer_body(head, p_ref, cnt_ref, h_ref, ce_ref, cl_ref,
                w1_ref, b1_ref, w2_ref, b2_ref, g_ref, bt_ref,
                wm1_ref, bm1_ref, wm2_ref, bm2_ref, wm3_ref, bm3_ref, o_ref):
    cnt = cnt_ref[0, :N, :] + cnt_ref[1, :N, :]
    agg = (p_ref[:N, :] + h_ref[...]
           + jnp.dot(cnt, ce_ref[...], preferred_element_type=jnp.float32)
           + cl_ref[...])
    hid = jnp.maximum(
        jnp.dot(agg, w1_ref[...], preferred_element_type=jnp.float32)
        + b1_ref[...], 0.0)
    hn = (jnp.dot(hid, w2_ref[...], preferred_element_type=jnp.float32)
          + b2_ref[...])
    mean = jnp.mean(hn, axis=0, keepdims=True)
    var = jnp.mean((hn - mean) ** 2, axis=0, keepdims=True)
    hn = (hn - mean) * lax.rsqrt(var + 1e-5) * g_ref[...] + bt_ref[...]
    if head:
        z = jnp.maximum(
            jnp.dot(hn, wm1_ref[...], preferred_element_type=jnp.float32)
            + bm1_ref[...], 0.0)
        z = jnp.maximum(
            jnp.dot(z, wm2_ref[...], preferred_element_type=jnp.float32)
            + bm2_ref[...], 0.0)
        logit = (jnp.dot(z, wm3_ref[...], preferred_element_type=jnp.float32)
                 + bm3_ref[...])
        o_ref[...] = jax.nn.sigmoid(logit)
    else:
        o_ref[...] = jnp.maximum(hn, 0.0)


_layer_tc = pl.pallas_call(
    functools.partial(_layer_body, False),
    out_shape=jax.ShapeDtypeStruct((N, EMB), jnp.float32))

_layer_tc_head = pl.pallas_call(
    functools.partial(_layer_body, True),
    out_shape=jax.ShapeDtypeStruct((N, 1), jnp.float32))


def kernel(x, edge_index, edge_attr, atom_emb1, atom_emb2, W1, b1, W2, b2,
           bond_emb, bond_dir_emb, bn_gamma, bn_beta,
           Wm1, bm1, Wm2, bm2, Wm3, bm3):
    f32 = jnp.float32
    src = edge_index[0].astype(jnp.int32)
    dst = edge_index[1].astype(jnp.int32)
    combo = (edge_attr[:, 0] * 3 + edge_attr[:, 1]).astype(jnp.int32)
    pad = EPAD - E
    src_p = jnp.concatenate([src, jnp.zeros((pad,), jnp.int32)]
                            ).reshape(EPAD // K, K)
    dst_p = jnp.concatenate([dst, jnp.full((pad,), NPAD - 1, jnp.int32)]
                            ).reshape(EPAD // K, K)
    combo_p = jnp.concatenate([combo, jnp.zeros((pad,), jnp.int32)]
                              ).reshape(EPAD // K, K)
    combo_p = combo_p + 16 * jnp.arange(K, dtype=jnp.int32)[None, :]
    id_rep = jnp.tile(jnp.eye(16, HC, dtype=f32), (K, 1))
    z_agg = jnp.zeros((ROWS_PER_TILE, HC), f32)

    oh_iota = jnp.arange(8, dtype=x.dtype)[None, :]
    x0h = (x[:, 0:1] == oh_iota).astype(f32)
    x1h = (x[:, 1:2] == oh_iota).astype(f32)
    a1p = jnp.zeros((8, EMB), f32).at[:3].set(atom_emb1[:3])
    a2p = jnp.zeros((8, EMB), f32).at[:3].set(atom_emb2[:3])
    h = _init_tc(x0h, x1h, a1p, a2p)

    cnts = _seg_es(id_rep, combo_p, dst_p, z_agg)

    ia = jnp.repeat(jnp.arange(3), 3)
    ib = jnp.tile(jnp.arange(3), 3)
    b1_2d = b1.reshape(NUM_LAYERS, 1, 2 * EMB)
    b2_2d = b2.reshape(NUM_LAYERS, 1, EMB)
    g_2d = bn_gamma.reshape(NUM_LAYERS, 1, EMB)
    bt_2d = bn_beta.reshape(NUM_LAYERS, 1, EMB)
    bm1_2d = bm1.reshape(1, 2 * EMB)
    bm2_2d = bm2.reshape(1, EMB)
    bm3_2d = bm3.reshape(1, 1)

    out = None
    for l in range(NUM_LAYERS):
        parts = _seg(h, src_p, dst_p, z_agg)
        ce_l = jnp.zeros((HC, EMB), f32).at[:9].set(
            bond_emb[l][ia] + bond_dir_emb[l][ib])
        cl = (bond_emb[l][4] + bond_dir_emb[l][0]).reshape(1, EMB)
        fn = _layer_tc if l < NUM_LAYERS - 1 else _layer_tc_head
        out = fn(parts, cnts, h, ce_l, cl,
                 W1[l], b1_2d[l], W2[l], b2_2d[l], g_2d[l], bt_2d[l],
                 Wm1, bm1_2d, Wm2, bm2_2d, Wm3, bm3_2d)
        if l < NUM_LAYERS - 1:
            h = out
    return out.reshape(-1)


# speedup vs baseline: 1.1192x; 1.0046x over previous
import functools

import jax
import jax.numpy as jnp
from jax import lax
from jax.experimental import pallas as pl
from jax.experimental.pallas import tpu as pltpu
from jax.experimental.pallas import tpu_sc as plsc

NUM_LAYERS = 5
EMB = 128
HC = 64
N = 10000
E = 320000

NC = 2
NS = 16
K = 128
NPAD = 10240
ROWS_PER_TILE = NPAD // NS
EPAD = 327680
ET = EPAD // NS
NCHUNK = ET // K
NBUF = 4

_MESH = plsc.VectorSubcoreMesh(
    core_axis_name="c", subcore_axis_name="s", num_cores=NC, num_subcores=NS)


def _seg_body(edge_split, tab_hbm, ind_hbm, dst_hbm, z_hbm, out_hbm,
              src_t, dst_t, rows_v, *rest):
    nch = NCHUNK // 2 if edge_split else NCHUNK
    gsem = rest[0:NBUF]
    ssem = rest[NBUF:2 * NBUF]
    agg_sh = rest[2 * NBUF]
    c = lax.axis_index("c")
    s = lax.axis_index("s")
    r0 = s * ROWS_PER_TILE
    pltpu.sync_copy(z_hbm, agg_sh.at[pl.ds(r0, ROWS_PER_TILE)])

    cbase = ((c * NS + s) * nch) if edge_split else (s * nch)
    pltpu.sync_copy(ind_hbm.at[pl.ds(cbase, nch)], src_t)
    pltpu.sync_copy(dst_hbm.at[pl.ds(cbase, nch)], dst_t)
    plsc.subcore_barrier()

    tab_c = tab_hbm if edge_split else tab_hbm.at[c]

    def fire(i, b):
        pltpu.async_copy(tab_c.at[src_t.at[i]], rows_v.at[b], gsem[b])

    for b in range(NBUF):
        fire(b, b)

    def outer(j, carry):
        base_i = j * NBUF
        for b in range(NBUF):
            i = base_i + b
            pltpu.make_async_copy(
                tab_c.at[src_t.at[i]], rows_v.at[b], gsem[b]).wait()
            pltpu.async_copy(
                rows_v.at[b], agg_sh.at[dst_t.at[i]], ssem[b], add=True)
        for b in range(NBUF):
            i = base_i + b
            pltpu.make_async_copy(
                rows_v.at[b], agg_sh.at[dst_t.at[i]], ssem[b]).wait()
            nxt = base_i + NBUF + b

            @pl.when(nxt < nch)
            def _():
                fire(nxt, b)
        return carry

    lax.fori_loop(0, nch // NBUF, outer, 0)
    plsc.subcore_barrier()
    pltpu.sync_copy(agg_sh.at[pl.ds(r0, ROWS_PER_TILE)],
                    out_hbm.at[c, pl.ds(r0, ROWS_PER_TILE)])


def _make_seg(edge_split):
    nch = NCHUNK // 2 if edge_split else NCHUNK
    return functools.partial(
        pl.kernel, functools.partial(_seg_body, edge_split),
        out_type=jax.ShapeDtypeStruct((NC, NPAD, HC), jnp.float32),
        mesh=_MESH,
        scratch_types=([pltpu.VMEM((nch, K), jnp.int32),
                        pltpu.VMEM((nch, K), jnp.int32),
                        pltpu.VMEM((NBUF, K, HC), jnp.float32)]
                       + [pltpu.SemaphoreType.DMA] * (2 * NBUF)
                       + [pltpu.VMEM_SHARED((NPAD, HC), jnp.float32)]),
        compiler_params=pltpu.CompilerParams(use_tc_tiling_on_sc=False),
    )()


_seg_es = _make_seg(True)

TROWS = N // NS
NPH = 4
PCH = NCHUNK // NPH


def _seg_sp_body(tab_hbm, ind_hbm, dst_hbm, z_hbm, out_hbm,
                 src_t, dst_t, rows_v, *rest):
    gsem = rest[0:NBUF]
    ssem = rest[NBUF:2 * NBUF]
    agg_sh = rest[2 * NBUF]
    tab_sh = rest[2 * NBUF + 1]
    c = lax.axis_index("c")
    s = lax.axis_index("s")
    r0 = s * ROWS_PER_TILE
    t0 = s * TROWS
    pltpu.sync_copy(tab_hbm.at[pl.ds(t0, TROWS), pl.ds(c * HC, HC)],
                    tab_sh.at[pl.ds(t0, TROWS)])
    pltpu.sync_copy(tab_hbm.at[pl.ds(t0, TROWS), pl.ds(c * HC, HC)],
                    agg_sh.at[pl.ds(t0, TROWS)])

    @pl.when(s == NS - 1)
    def _():
        pltpu.sync_copy(z_hbm.at[pl.ds(0, NPAD - N)],
                        agg_sh.at[pl.ds(N, NPAD - N)])

    plsc.subcore_barrier()

    def fire(i, b):
        pltpu.async_copy(tab_sh.at[src_t.at[i]], rows_v.at[b], gsem[b])

    def outer(j, carry):
        base_i = j * NBUF
        for b in range(NBUF):
            i = base_i + b
            pltpu.make_async_copy(
                tab_sh.at[src_t.at[i]], rows_v.at[b], gsem[b]).wait()
            pltpu.async_copy(
                rows_v.at[b], agg_sh.at[dst_t.at[i]], ssem[b], add=True)
        for b in range(NBUF):
            i = base_i + b
            pltpu.make_async_copy(
                rows_v.at[b], agg_sh.at[dst_t.at[i]], ssem[b]).wait()
            nxt = base_i + NBUF + b

            @pl.when(nxt < PCH)
            def _():
                fire(nxt, b)
        return carry

    for ph in range(NPH):
        cbase = s * NCHUNK + ph * PCH
        pltpu.sync_copy(ind_hbm.at[pl.ds(cbase, PCH)], src_t)
        pltpu.sync_copy(dst_hbm.at[pl.ds(cbase, PCH)], dst_t)
        for b in range(NBUF):
            fire(b, b)
        lax.fori_loop(0, PCH // NBUF, outer, 0)

    plsc.subcore_barrier()
    pltpu.sync_copy(agg_sh.at[pl.ds(r0, ROWS_PER_TILE)],
                    out_hbm.at[pl.ds(r0, ROWS_PER_TILE), pl.ds(c * HC, HC)])


_seg = functools.partial(
    pl.kernel, _seg_sp_body,
    out_type=jax.ShapeDtypeStruct((NPAD, EMB), jnp.float32),
    mesh=_MESH,
    scratch_types=([pltpu.VMEM((PCH, K), jnp.int32),
                    pltpu.VMEM((PCH, K), jnp.int32),
                    pltpu.VMEM((NBUF, K, HC), jnp.float32)]
                   + [pltpu.SemaphoreType.DMA] * (2 * NBUF)
                   + [pltpu.VMEM_SHARED((NPAD, HC), jnp.float32),
                      pltpu.VMEM_SHARED((N, HC), jnp.float32)]),
    compiler_params=pltpu.CompilerParams(use_tc_tiling_on_sc=False),
)()


def _init_body(x0h_ref, x1h_ref, a1_ref, a2_ref, o_ref):
    o_ref[...] = (
        jnp.dot(x0h_ref[...], a1_ref[...], preferred_element_type=jnp.float32)
        + jnp.dot(x1h_ref[...], a2_ref[...], preferred_element_type=jnp.float32))


_init_tc = pl.pallas_call(
    _init_body, out_shape=jax.ShapeDtypeStruct((N, EMB), jnp.float32))


def _layer_body(head, p_ref, cnt_ref, ce_ref, cl_ref,
                w1_ref, b1_ref, w2_ref, b2_ref, g_ref, bt_ref,
                wm1_ref, bm1_ref, wm2_ref, bm2_ref, wm3_ref, bm3_ref, o_ref):
    cnt = cnt_ref[0, :N, :] + cnt_ref[1, :N, :]
    agg = (p_ref[:N, :]
           + jnp.dot(cnt, ce_ref[...], preferred_element_type=jnp.float32)
           + cl_ref[...])
    hid = jnp.maximum(
        jnp.dot(agg, w1_ref[...], preferred_element_type=jnp.float32)
        + b1_ref[...], 0.0)
    hn = (jnp.dot(hid, w2_ref[...], preferred_element_type=jnp.float32)
          + b2_ref[...])
    mean = jnp.mean(hn, axis=0, keepdims=True)
    var = jnp.mean((hn - mean) ** 2, axis=0, keepdims=True)
    hn = (hn - mean) * lax.rsqrt(var + 1e-5) * g_ref[...] + bt_ref[...]
    if head:
        z = jnp.maximum(
            jnp.dot(hn, wm1_ref[...], preferred_element_type=jnp.float32)
            + bm1_ref[...], 0.0)
        z = jnp.maximum(
            jnp.dot(z, wm2_ref[...], preferred_element_type=jnp.float32)
            + bm2_ref[...], 0.0)
        logit = (jnp.dot(z, wm3_ref[...], preferred_element_type=jnp.float32)
                 + bm3_ref[...])
        o_ref[...] = jax.nn.sigmoid(logit)
    else:
        o_ref[...] = jnp.maximum(hn, 0.0)


_layer_tc = pl.pallas_call(
    functools.partial(_layer_body, False),
    out_shape=jax.ShapeDtypeStruct((N, EMB), jnp.float32))

_layer_tc_head = pl.pallas_call(
    functools.partial(_layer_body, True),
    out_shape=jax.ShapeDtypeStruct((N, 1), jnp.float32))


def kernel(x, edge_index, edge_attr, atom_emb1, atom_emb2, W1, b1, W2, b2,
           bond_emb, bond_dir_emb, bn_gamma, bn_beta,
           Wm1, bm1, Wm2, bm2, Wm3, bm3):
    f32 = jnp.float32
    src = edge_index[0].astype(jnp.int32)
    dst = edge_index[1].astype(jnp.int32)
    combo = (edge_attr[:, 0] * 3 + edge_attr[:, 1]).astype(jnp.int32)
    pad = EPAD - E
    src_p = jnp.concatenate([src, jnp.zeros((pad,), jnp.int32)]
                            ).reshape(EPAD // K, K)
    dst_p = jnp.concatenate([dst, jnp.full((pad,), NPAD - 1, jnp.int32)]
                            ).reshape(EPAD // K, K)
    combo_p = jnp.concatenate([combo, jnp.zeros((pad,), jnp.int32)]
                              ).reshape(EPAD // K, K)
    combo_p = combo_p + 16 * jnp.arange(K, dtype=jnp.int32)[None, :]
    id_rep = jnp.tile(jnp.eye(16, HC, dtype=f32), (K, 1))
    z_agg = jnp.zeros((ROWS_PER_TILE, HC), f32)

    oh_iota = jnp.arange(8, dtype=x.dtype)[None, :]
    x0h = (x[:, 0:1] == oh_iota).astype(f32)
    x1h = (x[:, 1:2] == oh_iota).astype(f32)
    a1p = jnp.zeros((8, EMB), f32).at[:3].set(atom_emb1[:3])
    a2p = jnp.zeros((8, EMB), f32).at[:3].set(atom_emb2[:3])
    h = _init_tc(x0h, x1h, a1p, a2p)

    cnts = _seg_es(id_rep, combo_p, dst_p, z_agg)

    ia = jnp.repeat(jnp.arange(3), 3)
    ib = jnp.tile(jnp.arange(3), 3)
    b1_2d = b1.reshape(NUM_LAYERS, 1, 2 * EMB)
    b2_2d = b2.reshape(NUM_LAYERS, 1, EMB)
    g_2d = bn_gamma.reshape(NUM_LAYERS, 1, EMB)
    bt_2d = bn_beta.reshape(NUM_LAYERS, 1, EMB)
    bm1_2d = bm1.reshape(1, 2 * EMB)
    bm2_2d = bm2.reshape(1, EMB)
    bm3_2d = bm3.reshape(1, 1)

    out = None
    for l in range(NUM_LAYERS):
        parts = _seg(h, src_p, dst_p, z_agg)
        ce_l = jnp.zeros((HC, EMB), f32).at[:9].set(
            bond_emb[l][ia] + bond_dir_emb[l][ib])
        cl = (bond_emb[l][4] + bond_dir_emb[l][0]).reshape(1, EMB)
        fn = _layer_tc if l < NUM_LAYERS - 1 else _layer_tc_head
        out = fn(parts, cnts, ce_l, cl,
                 W1[l], b1_2d[l], W2[l], b2_2d[l], g_2d[l], bt_2d[l],
                 Wm1, bm1_2d, Wm2, bm2_2d, Wm3, bm3_2d)
        if l < NUM_LAYERS - 1:
            h = out
    return out.reshape(-1)
